# Initial kernel scaffold; baseline (speedup 1.0000x reference)
import jax
import jax.numpy as jnp
from jax.experimental import pallas as pl


def kernel(x, edge_index, batch, bn_gamma, bn_beta, W1, b1, W2, b2, W3, b3, W4, b4, lin_W, lin_b):
    raise NotImplementedError("write your pallas kernel here")


# speedup vs baseline: 5.8864x; 5.8864x over previous
import functools

import jax
import jax.numpy as jnp
from jax import lax
from jax.experimental import pallas as pl
from jax.experimental.pallas import tpu as pltpu
from jax.experimental.pallas import tpu_sc as plsc

N = 10000
D = 128
G = 64
C = 16
E = 320000

NC = 2
NS = 16
NW = NC * NS

CH = 128
EPW = 10240
NCHUNK = EPW // CH
E_PAD = EPW * NW
N_PAD = 10240
RPS = N_PAD // NS

_f32 = jnp.float32


def _tc_bn_matmul_body(x_ref, gam_ref, bet_ref, w_ref, y_ref):
    x = x_ref[...]
    mean = jnp.sum(x, axis=0, keepdims=True) * (1.0 / N)
    msq = jnp.sum(x * x, axis=0, keepdims=True) * (1.0 / N)
    var = msq - mean * mean
    rstd = lax.rsqrt(var + 1e-5)
    h0 = (x - mean) * (rstd * gam_ref[...]) + bet_ref[...]
    y_ref[...] = jnp.dot(h0, w_ref[...], preferred_element_type=_f32)


def _tc_scale_body(dacc_ref, y_ref, dinv_ref, g_ref):
    dacc = dacc_ref[...]
    deg = dacc[0, :, 0:1] + dacc[1, :, 0:1] + 1.0
    row = lax.broadcasted_iota(jnp.int32, (N_PAD, 1), 0)
    dinv = jnp.where(row < N, lax.rsqrt(deg), 0.0)
    dinv_ref[...] = dinv
    g_ref[...] = y_ref[...] * dinv


def _tc_mid_body(sacc_ref, g_ref, dinv_ref, b_ref, w_ref, gout_ref):
    dinv = dinv_ref[...]
    s = sacc_ref[0] + sacc_ref[1] - g_ref[...]
    h = jnp.maximum(dinv * s + b_ref[...], 0.0)
    gout_ref[...] = jnp.dot(h, w_ref[...], preferred_element_type=_f32) * dinv


def _tc_post_body(sacc_ref, g_ref, dinv_ref, b_ref, batch_ref, lw_ref, lb_ref,
                  out_ref):
    dinv = dinv_ref[...]
    s = sacc_ref[0] + sacc_ref[1] - g_ref[...]
    h4 = dinv * s + b_ref[...]
    seg = lax.broadcasted_iota(jnp.int32, (N_PAD, G), 1)
    onehot = (batch_ref[...] == seg).astype(_f32)
    sums = lax.dot_general(onehot, h4, (((0,), (0,)), ((), ())),
                           preferred_element_type=_f32)
    cnt = jnp.sum(onehot, axis=0)[:, None]
    pooled = sums / jnp.maximum(cnt, 1.0)
    out_ref[...] = jnp.dot(pooled, lw_ref[...], preferred_element_type=_f32) \
        + lb_ref[...]


def _tc_call(body, out_shape, *args):
    return pl.pallas_call(body, out_shape=out_shape)(*args)


_MESH = plsc.VectorSubcoreMesh(core_axis_name="c", subcore_axis_name="s")


@functools.partial(
    pl.kernel, mesh=_MESH,
    out_type=jax.ShapeDtypeStruct((NC, N_PAD, D), _f32),
    scratch_types=[
        pltpu.VMEM_SHARED((N_PAD, D), _f32),
        pltpu.VMEM((NCHUNK, CH), jnp.int32),
        pltpu.VMEM((CH, D), _f32),
    ],
)
def _sc_deg(dst3_hbm, ones_hbm, zer_hbm, out_hbm, acc, dst_v, ones_v):
    c = lax.axis_index("c")
    s = lax.axis_index("s")
    wid = s * NC + c
    r0 = s * RPS
    pltpu.sync_copy(ones_hbm, ones_v)
    pltpu.sync_copy(dst3_hbm.at[wid], dst_v)
    pltpu.sync_copy(zer_hbm.at[pl.ds(r0, RPS)], acc.at[pl.ds(r0, RPS)])
    plsc.subcore_barrier()

    @pl.loop(0, NCHUNK)
    def _(j):
        pltpu.sync_copy(ones_v, acc.at[dst_v.at[j]], add=True)

    plsc.subcore_barrier()
    pltpu.sync_copy(acc.at[pl.ds(r0, RPS)], out_hbm.at[c].at[pl.ds(r0, RPS)])


@functools.partial(
    pl.kernel, mesh=_MESH,
    out_type=jax.ShapeDtypeStruct((NC, N_PAD, D), _f32),
    scratch_types=[
        pltpu.VMEM_SHARED((N_PAD, D), _f32),
        pltpu.VMEM((NCHUNK, CH), jnp.int32),
        pltpu.VMEM((NCHUNK, CH), jnp.int32),
        pltpu.VMEM((CH, D), _f32),
        pltpu.SemaphoreType.DMA,
    ],
)
def _sc_scatter(g_hbm, src3_hbm, dst3_hbm, out_hbm, acc, src_v, dst_v, rows,
                sem):
    c = lax.axis_index("c")
    s = lax.axis_index("s")
    wid = s * NC + c
    r0 = s * RPS
    pltpu.sync_copy(src3_hbm.at[wid], src_v)
    pltpu.sync_copy(dst3_hbm.at[wid], dst_v)
    pltpu.sync_copy(g_hbm.at[pl.ds(r0, RPS)], acc.at[pl.ds(r0, RPS)])
    plsc.subcore_barrier()

    @pl.loop(0, NCHUNK)
    def _(j):
        pltpu.async_copy(g_hbm.at[src_v.at[j]], rows, sem).wait()
        pltpu.sync_copy(rows, acc.at[dst_v.at[j]], add=True)

    plsc.subcore_barrier()
    pltpu.sync_copy(acc.at[pl.ds(r0, RPS)], out_hbm.at[c].at[pl.ds(r0, RPS)])


def kernel(x, edge_index, batch, bn_gamma, bn_beta, W1, b1, W2, b2, W3, b3,
           W4, b4, lin_W, lin_b):
    x_pad = jnp.zeros((N_PAD, D), _f32).at[:N].set(x)
    src = edge_index[0]
    dst = edge_index[1]
    pad = E_PAD - E
    src3 = jnp.concatenate([src, jnp.zeros((pad,), jnp.int32)]) \
        .reshape(NW, NCHUNK, CH)
    dst3 = jnp.concatenate([dst, jnp.full((pad,), N, jnp.int32)]) \
        .reshape(NW, NCHUNK, CH)
    batch2 = jnp.concatenate([batch, jnp.full((N_PAD - N,), G, jnp.int32)]) \
        .reshape(N_PAD, 1)
    ones16 = jnp.ones((CH, D), _f32)
    zer16 = jnp.zeros((N_PAD, D), _f32)
    gam = bn_gamma.reshape(1, D)
    bet = bn_beta.reshape(1, D)
    b1r, b2r, b3r, b4r = (b.reshape(1, D) for b in (b1, b2, b3, b4))
    lbr = lin_b.reshape(1, C)

    dacc = _sc_deg(dst3, ones16, zer16)
    y1 = _tc_call(_tc_bn_matmul_body,
                  jax.ShapeDtypeStruct((N_PAD, D), _f32),
                  x_pad, gam, bet, W1)
    dinv, g = _tc_call(_tc_scale_body,
                       (jax.ShapeDtypeStruct((N_PAD, 1), _f32),
                        jax.ShapeDtypeStruct((N_PAD, D), _f32)),
                       dacc, y1)

    for (b_r, W_next) in ((b1r, W2), (b2r, W3), (b3r, W4)):
        sacc = _sc_scatter(g, src3, dst3)
        g = _tc_call(_tc_mid_body,
                     jax.ShapeDtypeStruct((N_PAD, D), _f32),
                     sacc, g, dinv, b_r, W_next)

    sacc = _sc_scatter(g, src3, dst3)
    out = _tc_call(_tc_post_body,
                   jax.ShapeDtypeStruct((G, C), _f32),
                   sacc, g, dinv, b4r, batch2, lin_W, lbr)
    return out
